# Initial kernel scaffold; baseline (speedup 1.0000x reference)
import jax
import jax.numpy as jnp
from jax.experimental import pallas as pl


def kernel(text, emb, W_ih_l0, W_hh_l0, b_ih_l0, b_hh_l0, W_ih_l0r, W_hh_l0r, b_ih_l0r, b_hh_l0r, W_ih_l1, W_hh_l1, b_ih_l1, b_hh_l1, W_ih_l1r, W_hh_l1r, b_ih_l1r, b_hh_l1r, fc_W, fc_b):
    raise NotImplementedError("write your pallas kernel here")


# speedup vs baseline: 1.4239x; 1.4239x over previous
import jax
import jax.numpy as jnp
from jax import lax
from jax.experimental import pallas as pl
from jax.experimental.pallas import tpu as pltpu
from jax.experimental.pallas import tpu_sc as plsc

T = 50
B = 1024
EMB = 100
EMBP = 128
HID = 128
OUT = 2
BC = 256
GW = 128


def _sc_gather(emb, idx):
    ni = idx.shape[1]
    mesh = plsc.VectorSubcoreMesh(core_axis_name="core",
                                  subcore_axis_name="subcore")

    @pl.kernel(out_type=jax.ShapeDtypeStruct((ni, EMBP), jnp.float32),
               mesh=mesh)
    def k(emb_hbm, idx_hbm, out_hbm):
        def body(i_vmem, o_vmem):
            pltpu.sync_copy(emb_hbm.at[i_vmem.at[0]], o_vmem)

        pltpu.emit_pipeline(
            body,
            grid=(ni // GW,),
            in_specs=[pl.BlockSpec((1, GW), index_map=lambda i: (0, i))],
            out_specs=[pl.BlockSpec((GW, EMBP),
                                    index_map=lambda i: (i, 0))],
            core_axis_name=("core", "subcore"),
            dimension_semantics=(pltpu.PARALLEL,),
        )(idx_hbm, out_hbm)

    return k(emb, idx)


def _lstm_body(x_ref, wih0f_ref, whh0f_ref, b0f_ref, wih0r_ref, whh0r_ref,
               b0r_ref, wih1f_ref, whh1f_ref, b1f_ref, wih1r_ref, whh1r_ref,
               b1r_ref, fcwf_ref, fcwr_ref, fcb_ref, out_ref,
               ys0_ref, h_ref, c_ref, hf_ref):
    f32 = jnp.float32

    def gates(xt, h, wih_ref, whh_ref, b_ref, c):
        g = (jnp.dot(xt, wih_ref[...], preferred_element_type=f32)
             + jnp.dot(h, whh_ref[...], preferred_element_type=f32)
             + b_ref[...])
        i = jax.nn.sigmoid(g[:, 0:HID])
        f = jax.nn.sigmoid(g[:, HID:2 * HID])
        gg = jnp.tanh(g[:, 2 * HID:3 * HID])
        o = jax.nn.sigmoid(g[:, 3 * HID:4 * HID])
        c = f * c + i * gg
        h = o * jnp.tanh(c)
        return h, c

    def scan(get_x, wih_ref, whh_ref, b_ref, reverse, store):
        h_ref[...] = jnp.zeros((BC, HID), f32)
        c_ref[...] = jnp.zeros((BC, HID), f32)

        def step(i, _):
            t = (T - 1 - i) if reverse else i
            h, c = gates(get_x(t), h_ref[...], wih_ref, whh_ref, b_ref,
                         c_ref[...])
            h_ref[...] = h
            c_ref[...] = c
            if store is not None:
                store(t, h)
            return 0

        lax.fori_loop(0, T, step, 0)

    scan(lambda t: x_ref[t], wih0f_ref, whh0f_ref, b0f_ref, False,
         lambda t, h: ys0_ref.__setitem__((t, slice(None), slice(0, HID)), h))
    scan(lambda t: x_ref[t], wih0r_ref, whh0r_ref, b0r_ref, True,
         lambda t, h: ys0_ref.__setitem__(
             (t, slice(None), slice(HID, 2 * HID)), h))

    scan(lambda t: ys0_ref[t], wih1f_ref, whh1f_ref, b1f_ref, False, None)
    hf_ref[...] = h_ref[...]
    scan(lambda t: ys0_ref[t], wih1r_ref, whh1r_ref, b1r_ref, True, None)

    out_ref[...] = (
        jnp.dot(hf_ref[...], fcwf_ref[...], preferred_element_type=f32)
        + jnp.dot(h_ref[...], fcwr_ref[...], preferred_element_type=f32)
        + fcb_ref[...])


def _tc_bilstm(x, wih0f, whh0f, b0f, wih0r, whh0r, b0r,
               wih1f, whh1f, b1f, wih1r, whh1r, b1r, fcwf, fcwr, fcb):
    f32 = jnp.float32
    full = lambda a: pl.BlockSpec(a.shape, lambda i: (0,) * a.ndim)
    return pl.pallas_call(
        _lstm_body,
        grid=(B // BC,),
        in_specs=[
            pl.BlockSpec((T, BC, EMBP), lambda i: (0, i, 0)),
            full(wih0f), full(whh0f), full(b0f),
            full(wih0r), full(whh0r), full(b0r),
            full(wih1f), full(whh1f), full(b1f),
            full(wih1r), full(whh1r), full(b1r),
            full(fcwf), full(fcwr), full(fcb),
        ],
        out_specs=pl.BlockSpec((BC, OUT), lambda i: (i, 0)),
        out_shape=jax.ShapeDtypeStruct((B, OUT), f32),
        scratch_shapes=[
            pltpu.VMEM((T, BC, 2 * HID), f32),
            pltpu.VMEM((BC, HID), f32),
            pltpu.VMEM((BC, HID), f32),
            pltpu.VMEM((BC, HID), f32),
        ],
        compiler_params=pltpu.CompilerParams(
            dimension_semantics=("arbitrary",)),
    )(x, wih0f, whh0f, b0f, wih0r, whh0r, b0r,
      wih1f, whh1f, b1f, wih1r, whh1r, b1r, fcwf, fcwr, fcb)


def kernel(text, emb, W_ih_l0, W_hh_l0, b_ih_l0, b_hh_l0, W_ih_l0r, W_hh_l0r,
           b_ih_l0r, b_hh_l0r, W_ih_l1, W_hh_l1, b_ih_l1, b_hh_l1, W_ih_l1r,
           W_hh_l1r, b_ih_l1r, b_hh_l1r, fc_W, fc_b):
    idx = jnp.transpose(text).reshape(1, B * T)
    emb_p = jnp.pad(emb, ((0, 0), (0, EMBP - EMB)))
    x = _sc_gather(emb_p, idx).reshape(T, B, EMBP)
    zpad = ((0, EMBP - EMB), (0, 0))

    b0f = (b_ih_l0 + b_hh_l0).reshape(1, 4 * HID)
    b0r = (b_ih_l0r + b_hh_l0r).reshape(1, 4 * HID)
    b1f = (b_ih_l1 + b_hh_l1).reshape(1, 4 * HID)
    b1r = (b_ih_l1r + b_hh_l1r).reshape(1, 4 * HID)
    return _tc_bilstm(
        x,
        jnp.pad(W_ih_l0.T, zpad), W_hh_l0.T, b0f,
        jnp.pad(W_ih_l0r.T, zpad), W_hh_l0r.T, b0r,
        W_ih_l1.T, W_hh_l1.T, b1f,
        W_ih_l1r.T, W_hh_l1r.T, b1r,
        fc_W.T[0:HID], fc_W.T[HID:2 * HID], fc_b.reshape(1, OUT),
    )


# speedup vs baseline: 1.6248x; 1.1411x over previous
import jax
import jax.numpy as jnp
from jax import lax
from jax.experimental import pallas as pl
from jax.experimental.pallas import tpu as pltpu
from jax.experimental.pallas import tpu_sc as plsc

T = 50
B = 1024
EMB = 100
EMBP = 128
HID = 128
OUT = 2
BC = 512
GW = 128


def _sc_gather(emb, idx):
    ni = idx.shape[1]
    mesh = plsc.VectorSubcoreMesh(core_axis_name="core",
                                  subcore_axis_name="subcore")

    @pl.kernel(out_type=jax.ShapeDtypeStruct((ni, EMBP), jnp.float32),
               mesh=mesh)
    def k(emb_hbm, idx_hbm, out_hbm):
        def body(i_vmem, o_vmem):
            pltpu.sync_copy(emb_hbm.at[i_vmem.at[0]], o_vmem)

        pltpu.emit_pipeline(
            body,
            grid=(ni // GW,),
            in_specs=[pl.BlockSpec((1, GW), index_map=lambda i: (0, i))],
            out_specs=[pl.BlockSpec((GW, EMBP),
                                    index_map=lambda i: (i, 0))],
            core_axis_name=("core", "subcore"),
            dimension_semantics=(pltpu.PARALLEL,),
        )(idx_hbm, out_hbm)

    return k(emb, idx)


def _lstm_body(x_ref, wih0f_ref, whh0f_ref, b0f_ref, wih0r_ref, whh0r_ref,
               b0r_ref, wih1f_ref, whh1f_ref, b1f_ref, wih1r_ref, whh1r_ref,
               b1r_ref, fcwf_ref, fcwr_ref, fcb_ref, out_ref,
               ys0_ref, h_ref, c_ref, hf_ref):
    f32 = jnp.float32
    bf16 = jnp.bfloat16

    def gates(xt, h, wih_ref, whh_ref, b_ref, c):
        g = (jnp.dot(xt, wih_ref[...], preferred_element_type=f32)
             + jnp.dot(h, whh_ref[...], preferred_element_type=f32)
             + b_ref[...])
        i = jax.nn.sigmoid(g[:, 0:HID])
        f = jax.nn.sigmoid(g[:, HID:2 * HID])
        gg = jnp.tanh(g[:, 2 * HID:3 * HID])
        o = jax.nn.sigmoid(g[:, 3 * HID:4 * HID])
        c = f * c + i * gg
        h = o * jnp.tanh(c)
        return h, c

    def scan(get_x, wih_ref, whh_ref, b_ref, reverse, store):
        h_ref[...] = jnp.zeros((BC, HID), f32)
        c_ref[...] = jnp.zeros((BC, HID), f32)

        def step(i, _):
            t = (T - 1 - i) if reverse else i
            h, c = gates(get_x(t).astype(bf16), h_ref[...].astype(bf16),
                         wih_ref, whh_ref, b_ref, c_ref[...])
            h_ref[...] = h
            c_ref[...] = c
            if store is not None:
                store(t, h.astype(bf16))
            return 0

        lax.fori_loop(0, T, step, 0)

    scan(lambda t: x_ref[t], wih0f_ref, whh0f_ref, b0f_ref, False,
         lambda t, h: ys0_ref.__setitem__((t, slice(None), slice(0, HID)), h))
    scan(lambda t: x_ref[t], wih0r_ref, whh0r_ref, b0r_ref, True,
         lambda t, h: ys0_ref.__setitem__(
             (t, slice(None), slice(HID, 2 * HID)), h))

    scan(lambda t: ys0_ref[t], wih1f_ref, whh1f_ref, b1f_ref, False, None)
    hf_ref[...] = h_ref[...]
    scan(lambda t: ys0_ref[t], wih1r_ref, whh1r_ref, b1r_ref, True, None)

    out_ref[...] = (
        jnp.dot(hf_ref[...], fcwf_ref[...], preferred_element_type=f32)
        + jnp.dot(h_ref[...], fcwr_ref[...], preferred_element_type=f32)
        + fcb_ref[...])


def _tc_bilstm(x, wih0f, whh0f, b0f, wih0r, whh0r, b0r,
               wih1f, whh1f, b1f, wih1r, whh1r, b1r, fcwf, fcwr, fcb):
    f32 = jnp.float32
    bf16 = jnp.bfloat16
    full = lambda a: pl.BlockSpec(a.shape, lambda i: (0,) * a.ndim)
    return pl.pallas_call(
        _lstm_body,
        grid=(B // BC,),
        in_specs=[
            pl.BlockSpec((T, BC, EMBP), lambda i: (0, i, 0)),
            full(wih0f), full(whh0f), full(b0f),
            full(wih0r), full(whh0r), full(b0r),
            full(wih1f), full(whh1f), full(b1f),
            full(wih1r), full(whh1r), full(b1r),
            full(fcwf), full(fcwr), full(fcb),
        ],
        out_specs=pl.BlockSpec((BC, OUT), lambda i: (i, 0)),
        out_shape=jax.ShapeDtypeStruct((B, OUT), f32),
        scratch_shapes=[
            pltpu.VMEM((T, BC, 2 * HID), bf16),
            pltpu.VMEM((BC, HID), f32),
            pltpu.VMEM((BC, HID), f32),
            pltpu.VMEM((BC, HID), f32),
        ],
        compiler_params=pltpu.CompilerParams(
            dimension_semantics=("arbitrary",)),
    )(x, wih0f, whh0f, b0f, wih0r, whh0r, b0r,
      wih1f, whh1f, b1f, wih1r, whh1r, b1r, fcwf, fcwr, fcb)


def kernel(text, emb, W_ih_l0, W_hh_l0, b_ih_l0, b_hh_l0, W_ih_l0r, W_hh_l0r,
           b_ih_l0r, b_hh_l0r, W_ih_l1, W_hh_l1, b_ih_l1, b_hh_l1, W_ih_l1r,
           W_hh_l1r, b_ih_l1r, b_hh_l1r, fc_W, fc_b):
    bf16 = jnp.bfloat16
    idx = jnp.transpose(text).reshape(1, B * T)
    emb_p = jnp.pad(emb, ((0, 0), (0, EMBP - EMB)))
    x = _sc_gather(emb_p, idx).reshape(T, B, EMBP)

    b0f = (b_ih_l0 + b_hh_l0).reshape(1, 4 * HID)
    b0r = (b_ih_l0r + b_hh_l0r).reshape(1, 4 * HID)
    b1f = (b_ih_l1 + b_hh_l1).reshape(1, 4 * HID)
    b1r = (b_ih_l1r + b_hh_l1r).reshape(1, 4 * HID)
    zpad = ((0, EMBP - EMB), (0, 0))
    return _tc_bilstm(
        x,
        jnp.pad(W_ih_l0.T, zpad).astype(bf16), W_hh_l0.T.astype(bf16), b0f,
        jnp.pad(W_ih_l0r.T, zpad).astype(bf16), W_hh_l0r.T.astype(bf16), b0r,
        W_ih_l1.T.astype(bf16), W_hh_l1.T.astype(bf16), b1f,
        W_ih_l1r.T.astype(bf16), W_hh_l1r.T.astype(bf16), b1r,
        fc_W.T[0:HID], fc_W.T[HID:2 * HID], fc_b.reshape(1, OUT),
    )


# speedup vs baseline: 1.9268x; 1.1858x over previous
import jax
import jax.numpy as jnp
from jax import lax
from jax.experimental import pallas as pl
from jax.experimental.pallas import tpu as pltpu
from jax.experimental.pallas import tpu_sc as plsc

T = 50
B = 1024
EMB = 100
EMBP = 128
HID = 128
OUT = 2
BC = 1024
GW = 128


def _sc_gather(emb, idx):
    ni = idx.shape[1]
    mesh = plsc.VectorSubcoreMesh(core_axis_name="core",
                                  subcore_axis_name="subcore")

    @pl.kernel(out_type=jax.ShapeDtypeStruct((ni, EMBP), jnp.float32),
               mesh=mesh)
    def k(emb_hbm, idx_hbm, out_hbm):
        def body(i_vmem, o_vmem):
            pltpu.sync_copy(emb_hbm.at[i_vmem.at[0]], o_vmem)

        pltpu.emit_pipeline(
            body,
            grid=(ni // GW,),
            in_specs=[pl.BlockSpec((1, GW), index_map=lambda i: (0, i))],
            out_specs=[pl.BlockSpec((GW, EMBP),
                                    index_map=lambda i: (i, 0))],
            core_axis_name=("core", "subcore"),
            dimension_semantics=(pltpu.PARALLEL,),
        )(idx_hbm, out_hbm)

    return k(emb, idx)


def _sigmoid(x):
    return 0.5 * jnp.tanh(0.5 * x) + 0.5


def _lstm_body(x_ref, wih0f_ref, whh0f_ref, b0f_ref, wih0r_ref, whh0r_ref,
               b0r_ref, wih1f_ref, whh1f_ref, b1f_ref, wih1r_ref, whh1r_ref,
               b1r_ref, fcwf_ref, fcwr_ref, fcb_ref, out_ref,
               ys0_ref, hf_ref, cf_ref, hr_ref, cr_ref):
    f32 = jnp.float32
    bf16 = jnp.bfloat16

    def gates(xt, h, wih_ref, whh_ref, b_ref, c):
        g = (jnp.dot(xt, wih_ref[...], preferred_element_type=f32)
             + jnp.dot(h, whh_ref[...], preferred_element_type=f32)
             + b_ref[...])
        i = _sigmoid(g[:, 0:HID])
        f = _sigmoid(g[:, HID:2 * HID])
        gg = jnp.tanh(g[:, 2 * HID:3 * HID])
        o = _sigmoid(g[:, 3 * HID:4 * HID])
        c = f * c + i * gg
        h = o * jnp.tanh(c)
        return h, c

    def biscan(get_x, wihf_ref, whhf_ref, bf_ref, wihr_ref, whhr_ref, br_ref,
               store):
        z = jnp.zeros((BC, HID), f32)
        hf_ref[...] = z
        cf_ref[...] = z
        hr_ref[...] = z
        cr_ref[...] = z

        def step(i, _):
            tr = T - 1 - i
            hf, cf = gates(get_x(i), hf_ref[...].astype(bf16),
                           wihf_ref, whhf_ref, bf_ref, cf_ref[...])
            hr, cr = gates(get_x(tr), hr_ref[...].astype(bf16),
                           wihr_ref, whhr_ref, br_ref, cr_ref[...])
            hf_ref[...] = hf
            cf_ref[...] = cf
            hr_ref[...] = hr
            cr_ref[...] = cr
            if store:
                ys0_ref[i, :, 0:HID] = hf.astype(bf16)
                ys0_ref[tr, :, HID:2 * HID] = hr.astype(bf16)
            return 0

        lax.fori_loop(0, T, step, 0)

    biscan(lambda t: x_ref[t].astype(bf16), wih0f_ref, whh0f_ref, b0f_ref,
           wih0r_ref, whh0r_ref, b0r_ref, True)
    biscan(lambda t: ys0_ref[t], wih1f_ref, whh1f_ref, b1f_ref,
           wih1r_ref, whh1r_ref, b1r_ref, False)

    out_ref[...] = (
        jnp.dot(hf_ref[...], fcwf_ref[...], preferred_element_type=f32)
        + jnp.dot(hr_ref[...], fcwr_ref[...], preferred_element_type=f32)
        + fcb_ref[...])


def _tc_bilstm(x, wih0f, whh0f, b0f, wih0r, whh0r, b0r,
               wih1f, whh1f, b1f, wih1r, whh1r, b1r, fcwf, fcwr, fcb):
    f32 = jnp.float32
    bf16 = jnp.bfloat16
    full = lambda a: pl.BlockSpec(a.shape, lambda i: (0,) * a.ndim)
    return pl.pallas_call(
        _lstm_body,
        grid=(B // BC,),
        in_specs=[
            pl.BlockSpec((T, BC, EMBP), lambda i: (0, i, 0)),
            full(wih0f), full(whh0f), full(b0f),
            full(wih0r), full(whh0r), full(b0r),
            full(wih1f), full(whh1f), full(b1f),
            full(wih1r), full(whh1r), full(b1r),
            full(fcwf), full(fcwr), full(fcb),
        ],
        out_specs=pl.BlockSpec((BC, OUT), lambda i: (i, 0)),
        out_shape=jax.ShapeDtypeStruct((B, OUT), f32),
        scratch_shapes=[
            pltpu.VMEM((T, BC, 2 * HID), bf16),
            pltpu.VMEM((BC, HID), f32),
            pltpu.VMEM((BC, HID), f32),
            pltpu.VMEM((BC, HID), f32),
            pltpu.VMEM((BC, HID), f32),
        ],
        compiler_params=pltpu.CompilerParams(
            dimension_semantics=("arbitrary",)),
    )(x, wih0f, whh0f, b0f, wih0r, whh0r, b0r,
      wih1f, whh1f, b1f, wih1r, whh1r, b1r, fcwf, fcwr, fcb)


def kernel(text, emb, W_ih_l0, W_hh_l0, b_ih_l0, b_hh_l0, W_ih_l0r, W_hh_l0r,
           b_ih_l0r, b_hh_l0r, W_ih_l1, W_hh_l1, b_ih_l1, b_hh_l1, W_ih_l1r,
           W_hh_l1r, b_ih_l1r, b_hh_l1r, fc_W, fc_b):
    bf16 = jnp.bfloat16
    idx = jnp.transpose(text).reshape(1, B * T)
    emb_p = jnp.pad(emb, ((0, 0), (0, EMBP - EMB)))
    x = _sc_gather(emb_p, idx).reshape(T, B, EMBP)

    b0f = (b_ih_l0 + b_hh_l0).reshape(1, 4 * HID)
    b0r = (b_ih_l0r + b_hh_l0r).reshape(1, 4 * HID)
    b1f = (b_ih_l1 + b_hh_l1).reshape(1, 4 * HID)
    b1r = (b_ih_l1r + b_hh_l1r).reshape(1, 4 * HID)
    zpad = ((0, EMBP - EMB), (0, 0))
    return _tc_bilstm(
        x,
        jnp.pad(W_ih_l0.T, zpad).astype(bf16), W_hh_l0.T.astype(bf16), b0f,
        jnp.pad(W_ih_l0r.T, zpad).astype(bf16), W_hh_l0r.T.astype(bf16), b0r,
        W_ih_l1.T.astype(bf16), W_hh_l1.T.astype(bf16), b1f,
        W_ih_l1r.T.astype(bf16), W_hh_l1r.T.astype(bf16), b1r,
        fc_W.T[0:HID], fc_W.T[HID:2 * HID], fc_b.reshape(1, OUT),
    )
